# baseline (speedup 1.0000x reference)
import functools

import jax
import jax.numpy as jnp
from jax import lax
from jax.experimental import pallas as pl
from jax.experimental.pallas import tpu as pltpu
from jax.experimental.pallas import tpu_sc as plsc

VOCAB = 1000000
D = 64
B_TOTAL = 16384 * 50
NC, NS = 2, 16
NW = NC * NS
B_PER_W = B_TOTAL // NW
STEP = 128
N_STEPS = B_PER_W // STEP
NBUF = 4
SCALE = 8.0

_mesh = plsc.VectorSubcoreMesh(
    core_axis_name="c", subcore_axis_name="s", num_cores=NC, num_subcores=NS
)


@functools.partial(
    pl.kernel,
    out_type=jax.ShapeDtypeStruct((B_TOTAL, D), jnp.float32),
    mesh=_mesh,
    scratch_types=[
        pltpu.VMEM((N_STEPS, STEP), jnp.int32),
        pltpu.VMEM((NBUF, STEP, D), jnp.float32),
        pltpu.SemaphoreType.DMA,
        pltpu.SemaphoreType.DMA,
    ],
    compiler_params=pltpu.CompilerParams(use_tc_tiling_on_sc=False),
)
def _emb_kernel(idx_hbm, table_hbm, out_hbm, idx_v, bufs, gsem, ssem):
    wid = lax.axis_index("s") * NC + lax.axis_index("c")
    base = wid * B_PER_W

    pltpu.sync_copy(idx_hbm.at[wid], idx_v)

    for b in range(NBUF - 1):
        pltpu.async_copy(table_hbm.at[idx_v.at[b]], bufs.at[b], gsem)

    def do_step(g, b):
        buf = bufs.at[b]

        @pl.when(g >= 1)
        def _():
            pltpu.make_async_copy(
                bufs.at[(b - 1) % NBUF], out_hbm.at[pl.ds(0, STEP)], ssem
            ).wait()

        @pl.when(g + NBUF - 1 < N_STEPS)
        def _():
            pltpu.async_copy(
                table_hbm.at[idx_v.at[g + NBUF - 1]],
                bufs.at[(b + NBUF - 1) % NBUF],
                gsem,
            )

        pltpu.make_async_copy(table_hbm.at[pl.ds(0, STEP)], buf, gsem).wait()

        def mul_row(r, carry):
            for cg in range(D // 16):
                sl = pl.ds(cg * 16, 16)
                buf[r, sl] = buf[r, sl] * SCALE
            return carry

        lax.fori_loop(0, STEP, mul_row, 0, unroll=2)

        pltpu.async_copy(buf, out_hbm.at[pl.ds(base + g * STEP, STEP)], ssem)

    def outer(gg, carry):
        for b in range(NBUF):
            do_step(gg * NBUF + b, b)
        return carry

    lax.fori_loop(0, N_STEPS // NBUF, outer, 0)

    pltpu.make_async_copy(
        bufs.at[NBUF - 1], out_hbm.at[pl.ds(0, STEP)], ssem
    ).wait()


def kernel(x, table):
    idx = x.reshape(-1).astype(jnp.int32).reshape(NW, N_STEPS, STEP)
    out = _emb_kernel(idx, table)
    return out.reshape(x.shape[0], x.shape[1], D)
